# baseline (speedup 1.0000x reference)
import jax
import jax.numpy as jnp
from jax import lax
from jax.experimental import pallas as pl
from jax.experimental.pallas import tpu as pltpu
from jax.experimental.pallas import tpu_sc as plsc

N = 16384
B = 64
BH = B // 2
NC = 2
NT = 16
LANES = 16

STREAM = 128
NSTREAM = 3
CH = STREAM * NSTREAM
NBUF = 4
CHUNKS = 44
KT = CH * CHUNKS
K_TOTAL = KT * NT
GW = 32
GSTREAMS = CH // GW
PACK = 3 * NSTREAM + 1
ROWS_PER_TILE = N // NT


def _sc_body(x_hbm, pack_hbm, gl_hbm, zeros_hbm, out_hbm,
             ib0, ib1, ib2, ib3, gl0, gl1, gl2, gl3,
             gb0, gb1, gb2, gb3, cb0, cb1, acc,
             is0, is1, is2, is3, gs0, gs1, gs2, gs3, ss0, ss1):
    c = lax.axis_index("c")
    s = lax.axis_index("s")
    ibufs = [ib0, ib1, ib2, ib3]
    glbufs = [gl0, gl1, gl2, gl3]
    isems = [is0, is1, is2, is3]
    gbufs = [gb0, gb1, gb2, gb3]
    gsems = [gs0, gs1, gs2, gs3]
    cbufs = [cb0, cb1]
    ssems = [ss0, ss1]

    pltpu.sync_copy(zeros_hbm.at[pl.ds(s * ROWS_PER_TILE, ROWS_PER_TILE)],
                    acc.at[pl.ds(s * ROWS_PER_TILE, ROWS_PER_TILE)])
    plsc.subcore_barrier()

    def fire_idx(chunk, b):
        pltpu.async_copy(pack_hbm.at[s, chunk], ibufs[b], isems[b])
        pltpu.async_copy(gl_hbm.at[c, s, chunk], glbufs[b], isems[b])

    def drain_idx(b):
        pltpu.make_async_copy(pack_hbm.at[0, 0], ibufs[b], isems[b]).wait()
        pltpu.make_async_copy(gl_hbm.at[0, 0, 0], glbufs[b],
                              isems[b]).wait()

    def _cnt(b):
        return ibufs[b][3 * NSTREAM, pl.ds(0, LANES)][0]

    def fire_gather(b_ib, b):
        cnt = _cnt(b_ib)
        for j in range(GSTREAMS):
            @pl.when(cnt > j * GW)
            def _(j=j):
                pltpu.async_copy(
                    x_hbm.at[glbufs[b_ib].at[j]],
                    gbufs[b].at[pl.ds(j * GW, GW)], gsems[b])

    def drain_gather(b):
        cnt = _cnt(b)
        for j in range(GSTREAMS):
            @pl.when(cnt > j * GW)
            def _(j=j):
                pltpu.make_async_copy(
                    x_hbm.at[pl.ds(0, GW)],
                    gbufs[b].at[pl.ds(j * GW, GW)], gsems[b]).wait()

    def compute(b):
        gb = gbufs[b]
        ib = ibufs[b]
        cbuf = cbufs[b % 2]
        iota16 = lax.iota(jnp.int32, LANES)
        for j in range(NSTREAM):

            def grp(gg, carry, j=j, gb=gb, ib=ib, cbuf=cbuf):
                off = gg * LANES
                w16 = plsc.bitcast(
                    ib[NSTREAM + j, pl.ds(off, LANES)], jnp.float32)
                pos16 = ib[2 * NSTREAM + j, pl.ds(off, LANES)]
                rowv = iota16 + (j * STREAM + off)
                for l in range(BH):
                    lane = jnp.full((LANES,), l, jnp.int32)
                    vals = plsc.load_gather(gb, [pos16, lane])
                    plsc.store_scatter(cbuf, [rowv, lane], vals * w16)
                return carry

            lax.fori_loop(0, STREAM // LANES, grp, 0)

    def fire_scatter(b):
        cbuf = cbufs[b % 2]
        for j in range(NSTREAM):
            pltpu.async_copy(
                cbuf.at[pl.ds(j * STREAM, STREAM)],
                acc.at[ibufs[b].at[j]], ssems[b % 2], add=True)

    def drain_scatter(p):
        pltpu.make_async_copy(x_hbm.at[pl.ds(0, CH)], cbufs[p],
                              ssems[p]).wait()

    pltpu.sync_copy(pack_hbm.at[s, 0], ibufs[0])
    pltpu.sync_copy(gl_hbm.at[c, s, 0], glbufs[0])
    pltpu.sync_copy(pack_hbm.at[s, 1], ibufs[1])
    pltpu.sync_copy(gl_hbm.at[c, s, 1], glbufs[1])
    fire_idx(2, 2)
    fire_gather(0, 0)
    fire_gather(1, 1)

    def outer(c0, carry):
        for b in range(NBUF):
            ch = c0 + b
            drain_gather(b)
            compute(b)
            if b == 0:
                @pl.when(c0 > 0)
                def _():
                    drain_scatter((NBUF - 1) % 2)
            else:
                drain_scatter((b - 1) % 2)
            fire_scatter(b)
            g2 = (b + 2) % NBUF
            drain_idx(g2)
            fire_gather(g2, g2)
            i3 = (b + 3) % NBUF
            fire_idx(jnp.minimum(ch + 3, CHUNKS - 1), i3)
        return carry

    lax.fori_loop(0, CHUNKS // NBUF, lambda i, cr: outer(i * NBUF, cr), 0)

    drain_gather(0)
    drain_gather(1)
    drain_idx(2)
    drain_scatter((NBUF - 1) % 2)

    plsc.subcore_barrier()
    pltpu.sync_copy(acc.at[pl.ds(s * ROWS_PER_TILE, ROWS_PER_TILE)],
                    out_hbm.at[c, pl.ds(s * ROWS_PER_TILE, ROWS_PER_TILE)])


@jax.jit
def _sparse_matmul(xstack, pack, glist, zeros):
    mesh = plsc.VectorSubcoreMesh(core_axis_name="c", subcore_axis_name="s",
                                  num_cores=NC, num_subcores=NT)
    run = pl.kernel(
        _sc_body,
        out_type=jax.ShapeDtypeStruct((NC, N, BH), jnp.float32),
        mesh=mesh,
        scratch_types=(
            [pltpu.VMEM((PACK, STREAM), jnp.int32) for _ in range(NBUF)]
            + [pltpu.VMEM((GSTREAMS, GW), jnp.int32) for _ in range(NBUF)]
            + [pltpu.VMEM((CH, BH), jnp.float32) for _ in range(NBUF)]
            + [pltpu.VMEM((CH, BH), jnp.float32) for _ in range(2)]
            + [pltpu.VMEM_SHARED((N, BH), jnp.float32)]
            + [pltpu.SemaphoreType.DMA for _ in range(2 * NBUF + 2)]
        ),
        compiler_params=pltpu.CompilerParams(use_tc_tiling_on_sc=False,
                                             needs_layout_passes=False),
    )
    return run(xstack, pack, glist, zeros)


def kernel(inputs, w, indices):
    nnz = indices.shape[0]
    rows = indices[:, 0].astype(jnp.int32)
    cols = indices[:, 1].astype(jnp.int32)

    pad = K_TOTAL - nnz
    rows = jnp.pad(rows, (0, pad))
    cols = jnp.pad(cols, (0, pad))
    wp = jnp.pad(w.astype(jnp.float32), (0, pad))
    w_i = lax.bitcast_convert_type(wp, jnp.int32)

    xT = inputs.astype(jnp.float32).T
    xstack = jnp.concatenate([xT[:, :BH], xT[:, BH:]], axis=0)

    r = rows.reshape(NT, KT)
    first = jnp.concatenate(
        [jnp.ones((NT, 1), bool), r[:, 1:] != r[:, :-1]], axis=1)
    lid = jnp.cumsum(first.astype(jnp.int32), axis=1) - 1
    gf = jnp.zeros((NT, KT), jnp.int32).at[
        jnp.arange(NT)[:, None], lid].set(r)
    lidc = lid.reshape(NT, CHUNKS, CH)
    base = lidc[:, :, 0]
    pos = lidc - base[:, :, None]
    cnt = lidc[:, :, -1] - base + 1
    gidx = jnp.minimum(base[:, :, None] + jnp.arange(CH)[None, None, :],
                       KT - 1)
    glist = jnp.take_along_axis(
        jnp.broadcast_to(gf[:, None, :], (NT, CHUNKS, KT)), gidx,
        axis=2)

    def rows3(a):
        return a.reshape(NT, CHUNKS, NSTREAM, STREAM)

    cols_c = rows3(cols.reshape(NT, CHUNKS, CH))
    w_c = rows3(w_i.reshape(NT, CHUNKS, CH))
    pos_c = rows3(pos)
    cnt_c = jnp.broadcast_to(cnt[:, :, None, None],
                             (NT, CHUNKS, 1, STREAM)).astype(jnp.int32)
    pack = jnp.concatenate([cols_c, w_c, pos_c, cnt_c], axis=2)
    gl4 = glist.reshape(NT, CHUNKS, GSTREAMS, GW)
    glist_nc = jnp.stack([gl4 + ci * N for ci in range(NC)])

    zeros = jnp.zeros((N, BH), jnp.float32)

    o = _sparse_matmul(xstack, pack, glist_nc, zeros)
    return jnp.concatenate([o[0], o[1]], axis=1).T


# speedup vs baseline: 1.2281x; 1.2281x over previous
import jax
import jax.numpy as jnp
from jax import lax
from jax.experimental import pallas as pl
from jax.experimental.pallas import tpu as pltpu
from jax.experimental.pallas import tpu_sc as plsc

N = 16384
B = 64
BH = B // 2
NC = 2
NT = 16
LANES = 16

STREAM = 128
NSTREAM = 3
CH = STREAM * NSTREAM
NBUF = 4
CHUNKS = 44
KT = CH * CHUNKS
K_TOTAL = KT * NT
GW = 32
GSTREAMS = CH // GW
PACK = 3 * NSTREAM + 1
ROWS_PER_TILE = N // NT


def _sc_body(x_hbm, pack_hbm, gl_hbm, zeros_hbm, out_hbm,
             ib0, ib1, ib2, ib3, gl0, gl1, gl2, gl3,
             gb0, gb1, gb2, gb3, cb0, cb1, acc,
             is0, is1, is2, is3, gs0, gs1, gs2, gs3, ss0, ss1):
    c = lax.axis_index("c")
    s = lax.axis_index("s")
    ibufs = [ib0, ib1, ib2, ib3]
    glbufs = [gl0, gl1, gl2, gl3]
    isems = [is0, is1, is2, is3]
    gbufs = [gb0, gb1, gb2, gb3]
    gsems = [gs0, gs1, gs2, gs3]
    cbufs = [cb0, cb1]
    ssems = [ss0, ss1]

    pltpu.sync_copy(zeros_hbm.at[pl.ds(s * ROWS_PER_TILE, ROWS_PER_TILE)],
                    acc.at[pl.ds(s * ROWS_PER_TILE, ROWS_PER_TILE)])
    plsc.subcore_barrier()

    def fire_idx(chunk, b):
        pltpu.async_copy(pack_hbm.at[s, chunk], ibufs[b], isems[b])
        pltpu.async_copy(gl_hbm.at[c, s, chunk], glbufs[b], isems[b])

    def drain_idx(b):
        pltpu.make_async_copy(pack_hbm.at[0, 0], ibufs[b], isems[b]).wait()
        pltpu.make_async_copy(gl_hbm.at[0, 0, 0], glbufs[b],
                              isems[b]).wait()

    def _cnt(b):
        return ibufs[b][3 * NSTREAM, pl.ds(0, LANES)][0]

    def fire_gather(b_ib, b):
        cnt = _cnt(b_ib)
        for j in range(GSTREAMS):
            @pl.when(cnt > j * GW)
            def _(j=j):
                pltpu.async_copy(
                    x_hbm.at[glbufs[b_ib].at[j]],
                    gbufs[b].at[pl.ds(j * GW, GW)], gsems[b])

    def drain_gather(b):
        cnt = _cnt(b)
        for j in range(GSTREAMS):
            @pl.when(cnt > j * GW)
            def _(j=j):
                pltpu.make_async_copy(
                    x_hbm.at[pl.ds(0, GW)],
                    gbufs[b].at[pl.ds(j * GW, GW)], gsems[b]).wait()

    def compute(b):
        gb = gbufs[b]
        ib = ibufs[b]
        cbuf = cbufs[b % 2]
        for j in range(NSTREAM):

            def grp(gg, carry, j=j, gb=gb, ib=ib, cbuf=cbuf):
                off = gg * LANES
                w16 = plsc.bitcast(
                    ib[NSTREAM + j, pl.ds(off, LANES)], jnp.float32)
                pos16 = ib[2 * NSTREAM + j, pl.ds(off, LANES)]
                base = j * STREAM + off
                for k in range(LANES):
                    wk = w16[k]
                    pk = pos16[k]
                    r = base + k
                    cbuf[r, pl.ds(0, LANES)] = gb[pk, pl.ds(0, LANES)] * wk
                    cbuf[r, pl.ds(LANES, LANES)] = (
                        gb[pk, pl.ds(LANES, LANES)] * wk)
                return carry

            lax.fori_loop(0, STREAM // LANES, grp, 0)

    def fire_scatter(b):
        cbuf = cbufs[b % 2]
        for j in range(NSTREAM):
            pltpu.async_copy(
                cbuf.at[pl.ds(j * STREAM, STREAM)],
                acc.at[ibufs[b].at[j]], ssems[b % 2], add=True)

    def drain_scatter(p):
        pltpu.make_async_copy(x_hbm.at[pl.ds(0, CH)], cbufs[p],
                              ssems[p]).wait()

    pltpu.sync_copy(pack_hbm.at[s, 0], ibufs[0])
    pltpu.sync_copy(gl_hbm.at[c, s, 0], glbufs[0])
    pltpu.sync_copy(pack_hbm.at[s, 1], ibufs[1])
    pltpu.sync_copy(gl_hbm.at[c, s, 1], glbufs[1])
    fire_idx(2, 2)
    fire_gather(0, 0)
    fire_gather(1, 1)

    def outer(c0, carry):
        for b in range(NBUF):
            ch = c0 + b
            drain_gather(b)
            compute(b)
            if b == 0:
                @pl.when(c0 > 0)
                def _():
                    drain_scatter((NBUF - 1) % 2)
            else:
                drain_scatter((b - 1) % 2)
            fire_scatter(b)
            g2 = (b + 2) % NBUF
            drain_idx(g2)
            fire_gather(g2, g2)
            i3 = (b + 3) % NBUF
            fire_idx(jnp.minimum(ch + 3, CHUNKS - 1), i3)
        return carry

    lax.fori_loop(0, CHUNKS // NBUF, lambda i, cr: outer(i * NBUF, cr), 0)

    drain_gather(0)
    drain_gather(1)
    drain_idx(2)
    drain_scatter((NBUF - 1) % 2)

    plsc.subcore_barrier()
    pltpu.sync_copy(acc.at[pl.ds(s * ROWS_PER_TILE, ROWS_PER_TILE)],
                    out_hbm.at[c, pl.ds(s * ROWS_PER_TILE, ROWS_PER_TILE)])


@jax.jit
def _sparse_matmul(xstack, pack, glist, zeros):
    mesh = plsc.VectorSubcoreMesh(core_axis_name="c", subcore_axis_name="s",
                                  num_cores=NC, num_subcores=NT)
    run = pl.kernel(
        _sc_body,
        out_type=jax.ShapeDtypeStruct((NC, N, BH), jnp.float32),
        mesh=mesh,
        scratch_types=(
            [pltpu.VMEM((PACK, STREAM), jnp.int32) for _ in range(NBUF)]
            + [pltpu.VMEM((GSTREAMS, GW), jnp.int32) for _ in range(NBUF)]
            + [pltpu.VMEM((CH, BH), jnp.float32) for _ in range(NBUF)]
            + [pltpu.VMEM((CH, BH), jnp.float32) for _ in range(2)]
            + [pltpu.VMEM_SHARED((N, BH), jnp.float32)]
            + [pltpu.SemaphoreType.DMA for _ in range(2 * NBUF + 2)]
        ),
        compiler_params=pltpu.CompilerParams(use_tc_tiling_on_sc=False,
                                             needs_layout_passes=False),
    )
    return run(xstack, pack, glist, zeros)


def kernel(inputs, w, indices):
    nnz = indices.shape[0]
    rows = indices[:, 0].astype(jnp.int32)
    cols = indices[:, 1].astype(jnp.int32)

    pad = K_TOTAL - nnz
    rows = jnp.pad(rows, (0, pad))
    cols = jnp.pad(cols, (0, pad))
    wp = jnp.pad(w.astype(jnp.float32), (0, pad))
    w_i = lax.bitcast_convert_type(wp, jnp.int32)

    xT = inputs.astype(jnp.float32).T
    xstack = jnp.concatenate([xT[:, :BH], xT[:, BH:]], axis=0)

    r = rows.reshape(NT, KT)
    first = jnp.concatenate(
        [jnp.ones((NT, 1), bool), r[:, 1:] != r[:, :-1]], axis=1)
    lid = jnp.cumsum(first.astype(jnp.int32), axis=1) - 1
    sidx = jnp.where(first, lid, KT)
    gf = jnp.zeros((NT, KT), jnp.int32).at[
        jnp.arange(NT)[:, None], sidx].set(
            r, mode="drop", indices_are_sorted=True, unique_indices=True)
    lidc = lid.reshape(NT, CHUNKS, CH)
    base = lidc[:, :, 0]
    pos = lidc - base[:, :, None]
    cnt = lidc[:, :, -1] - base + 1
    gidx = jnp.minimum(base[:, :, None] + jnp.arange(CH)[None, None, :],
                       KT - 1).reshape(NT, KT)
    glist = jnp.take_along_axis(gf, gidx, axis=1).reshape(NT, CHUNKS, CH)

    def rows3(a):
        return a.reshape(NT, CHUNKS, NSTREAM, STREAM)

    cols_c = rows3(cols.reshape(NT, CHUNKS, CH))
    w_c = rows3(w_i.reshape(NT, CHUNKS, CH))
    pos_c = rows3(pos)
    cnt_c = jnp.broadcast_to(cnt[:, :, None, None],
                             (NT, CHUNKS, 1, STREAM)).astype(jnp.int32)
    pack = jnp.concatenate([cols_c, w_c, pos_c, cnt_c], axis=2)
    gl4 = glist.reshape(NT, CHUNKS, GSTREAMS, GW)
    glist_nc = jnp.stack([gl4 + ci * N for ci in range(NC)])

    zeros = jnp.zeros((N, BH), jnp.float32)

    o = _sparse_matmul(xstack, pack, glist_nc, zeros)
    return jnp.concatenate([o[0], o[1]], axis=1).T


# speedup vs baseline: 5.0070x; 4.0769x over previous
import functools

import jax
import jax.numpy as jnp
from jax import lax
from jax.experimental import pallas as pl
from jax.experimental.pallas import tpu as pltpu
from jax.experimental.pallas import tpu_sc as plsc

N = 16384
B = 64
BH = B // 2
NC = 2
NT = 16
LANES = 16

STREAM = 128
NSTREAM = 8
CH = STREAM * NSTREAM
CHUNKS = 17
KT = CH * CHUNKS
K_TOTAL = KT * NT
KROWS = K_TOTAL // STREAM
ROWS_PER_TILE = N // NT


def _sc_body(x_hbm, rows_hbm, cols_hbm, w_hbm, zeros_hbm, out_hbm,
             idx_r, idx_c, w_v, gath, acc, gsem, ssem):
    c = lax.axis_index("c")
    s = lax.axis_index("s")

    pltpu.sync_copy(zeros_hbm.at[pl.ds(s * ROWS_PER_TILE, ROWS_PER_TILE)],
                    acc.at[pl.ds(s * ROWS_PER_TILE, ROWS_PER_TILE)])
    plsc.subcore_barrier()

    def chunk_body(ch, carry):
        rowbase = s * (CHUNKS * NSTREAM) + ch * NSTREAM
        pltpu.sync_copy(rows_hbm.at[c, pl.ds(rowbase, NSTREAM)], idx_r)
        pltpu.sync_copy(cols_hbm.at[pl.ds(rowbase, NSTREAM)], idx_c)
        pltpu.sync_copy(w_hbm.at[pl.ds(rowbase, NSTREAM)], w_v)

        copies = []
        for j in range(NSTREAM):
            copies.append(pltpu.async_copy(
                x_hbm.at[idx_r.at[j]],
                gath.at[pl.ds(j * STREAM, STREAM)], gsem))
        for cp in copies:
            cp.wait()

        adds = []
        for j in range(NSTREAM):
            def grp(gg, carry2, j=j):
                w16 = w_v[j, pl.ds(gg * LANES, LANES)]
                base = j * STREAM + gg * LANES
                for k in range(LANES):
                    wk = w16[k]
                    r = base + k
                    gath[r, pl.ds(0, LANES)] = gath[r, pl.ds(0, LANES)] * wk
                    gath[r, pl.ds(LANES, LANES)] = (
                        gath[r, pl.ds(LANES, LANES)] * wk)
                return carry2
            lax.fori_loop(0, STREAM // LANES, grp, 0)
            adds.append(pltpu.async_copy(
                gath.at[pl.ds(j * STREAM, STREAM)],
                acc.at[idx_c.at[j]], ssem, add=True))
        for cp in adds:
            cp.wait()
        return carry

    lax.fori_loop(0, CHUNKS, chunk_body, 0)

    plsc.subcore_barrier()
    pltpu.sync_copy(acc.at[pl.ds(s * ROWS_PER_TILE, ROWS_PER_TILE)],
                    out_hbm.at[c, pl.ds(s * ROWS_PER_TILE, ROWS_PER_TILE)])


@jax.jit
def _sparse_matmul(xstack, rows2, cols_r, w_r, zeros):
    mesh = plsc.VectorSubcoreMesh(core_axis_name="c", subcore_axis_name="s",
                                  num_cores=NC, num_subcores=NT)
    run = pl.kernel(
        _sc_body,
        out_type=jax.ShapeDtypeStruct((NC, N, BH), jnp.float32),
        mesh=mesh,
        scratch_types=[
            pltpu.VMEM((NSTREAM, STREAM), jnp.int32),
            pltpu.VMEM((NSTREAM, STREAM), jnp.int32),
            pltpu.VMEM((NSTREAM, STREAM), jnp.float32),
            pltpu.VMEM((CH, BH), jnp.float32),
            pltpu.VMEM_SHARED((N, BH), jnp.float32),
            pltpu.SemaphoreType.DMA,
            pltpu.SemaphoreType.DMA,
        ],
        compiler_params=pltpu.CompilerParams(use_tc_tiling_on_sc=False),
    )
    return run(xstack, rows2, cols_r, w_r, zeros)


def kernel(inputs, w, indices):
    nnz = indices.shape[0]
    rows = indices[:, 0].astype(jnp.int32)
    cols = indices[:, 1].astype(jnp.int32)

    pad = K_TOTAL - nnz
    rows = jnp.pad(rows, (0, pad))
    cols = jnp.pad(cols, (0, pad))
    wp = jnp.pad(w.astype(jnp.float32), (0, pad))

    xT = inputs.astype(jnp.float32).T
    xstack = jnp.concatenate([xT[:, :BH], xT[:, BH:]], axis=0)

    rows2 = jnp.stack([rows, rows + N]).reshape(NC, KROWS, STREAM)
    cols_r = cols.reshape(KROWS, STREAM)
    w_r = wp.reshape(KROWS, STREAM)
    zeros = jnp.zeros((N, BH), jnp.float32)

    o = _sparse_matmul(xstack, rows2, cols_r, w_r, zeros)
    return jnp.concatenate([o[0], o[1]], axis=1).T


# speedup vs baseline: 5.1936x; 1.0373x over previous
import functools

import jax
import jax.numpy as jnp
from jax import lax
from jax.experimental import pallas as pl
from jax.experimental.pallas import tpu as pltpu
from jax.experimental.pallas import tpu_sc as plsc

N = 16384
B = 64
BH = B // 2
NC = 2
NT = 16
LANES = 16

STREAM = 128
NSTREAM = 8
CH = STREAM * NSTREAM
CHUNKS = 17
KT = CH * CHUNKS
K_TOTAL = KT * NT
KROWS = K_TOTAL // STREAM
ROWS_PER_TILE = N // NT


def _sc_body(x_hbm, rows_hbm, cols_hbm, w_hbm, zeros_hbm, out_hbm,
             idx_r, idx_c, w_v, gath, acc, gsem, ssem, rsem):
    c = lax.axis_index("c")
    s = lax.axis_index("s")

    pltpu.sync_copy(zeros_hbm.at[pl.ds(s * ROWS_PER_TILE, ROWS_PER_TILE)],
                    acc.at[pl.ds(s * ROWS_PER_TILE, ROWS_PER_TILE)])
    plsc.subcore_barrier()

    def chunk_body(ch, carry):
        rowbase = s * (CHUNKS * NSTREAM) + ch * NSTREAM
        pltpu.sync_copy(cols_hbm.at[pl.ds(rowbase, NSTREAM)], idx_c)
        pltpu.sync_copy(w_hbm.at[pl.ds(rowbase, NSTREAM)], w_v)
        pltpu.make_async_copy(rows_hbm.at[c, pl.ds(0, NSTREAM)], idx_r,
                              rsem).wait()

        copies = []
        for j in range(NSTREAM):
            copies.append(pltpu.async_copy(
                x_hbm.at[idx_r.at[j]],
                gath.at[pl.ds(j * STREAM, STREAM)], gsem))
        for cp in copies:
            cp.wait()

        nxtbase = s * (CHUNKS * NSTREAM) + jnp.minimum(ch + 1, CHUNKS - 1) * NSTREAM
        pltpu.async_copy(rows_hbm.at[c, pl.ds(nxtbase, NSTREAM)], idx_r,
                         rsem)

        adds = []
        for j in range(NSTREAM):
            def grp(gg, carry2, j=j):
                w16 = w_v[j, pl.ds(gg * LANES, LANES)]
                base = j * STREAM + gg * LANES
                for k in range(LANES):
                    wk = w16[k]
                    r = base + k
                    gath[r, pl.ds(0, LANES)] = gath[r, pl.ds(0, LANES)] * wk
                    gath[r, pl.ds(LANES, LANES)] = (
                        gath[r, pl.ds(LANES, LANES)] * wk)
                return carry2
            lax.fori_loop(0, STREAM // LANES, grp, 0)
            adds.append(pltpu.async_copy(
                gath.at[pl.ds(j * STREAM, STREAM)],
                acc.at[idx_c.at[j]], ssem, add=True))
        for cp in adds:
            cp.wait()
        return carry

    pltpu.async_copy(rows_hbm.at[c, pl.ds(s * (CHUNKS * NSTREAM), NSTREAM)],
                     idx_r, rsem)
    lax.fori_loop(0, CHUNKS, chunk_body, 0)
    pltpu.make_async_copy(rows_hbm.at[c, pl.ds(0, NSTREAM)], idx_r,
                          rsem).wait()

    plsc.subcore_barrier()
    pltpu.sync_copy(acc.at[pl.ds(s * ROWS_PER_TILE, ROWS_PER_TILE)],
                    out_hbm.at[c, pl.ds(s * ROWS_PER_TILE, ROWS_PER_TILE)])


@jax.jit
def _sparse_matmul(xstack, rows2, cols_r, w_r, zeros):
    mesh = plsc.VectorSubcoreMesh(core_axis_name="c", subcore_axis_name="s",
                                  num_cores=NC, num_subcores=NT)
    run = pl.kernel(
        _sc_body,
        out_type=jax.ShapeDtypeStruct((NC, N, BH), jnp.float32),
        mesh=mesh,
        scratch_types=[
            pltpu.VMEM((NSTREAM, STREAM), jnp.int32),
            pltpu.VMEM((NSTREAM, STREAM), jnp.int32),
            pltpu.VMEM((NSTREAM, STREAM), jnp.float32),
            pltpu.VMEM((CH, BH), jnp.float32),
            pltpu.VMEM_SHARED((N, BH), jnp.float32),
            pltpu.SemaphoreType.DMA,
            pltpu.SemaphoreType.DMA,
            pltpu.SemaphoreType.DMA,
        ],
        compiler_params=pltpu.CompilerParams(use_tc_tiling_on_sc=False),
    )
    return run(xstack, rows2, cols_r, w_r, zeros)


def kernel(inputs, w, indices):
    nnz = indices.shape[0]
    rows = indices[:, 0].astype(jnp.int32)
    cols = indices[:, 1].astype(jnp.int32)

    pad = K_TOTAL - nnz
    rows = jnp.pad(rows, (0, pad))
    cols = jnp.pad(cols, (0, pad))
    wp = jnp.pad(w.astype(jnp.float32), (0, pad))

    xT = inputs.astype(jnp.float32).T
    xstack = jnp.concatenate([xT[:, :BH], xT[:, BH:]], axis=0)

    rows2 = jnp.stack([rows, rows + N]).reshape(NC, KROWS, STREAM)
    cols_r = cols.reshape(KROWS, STREAM)
    w_r = wp.reshape(KROWS, STREAM)
    zeros = jnp.zeros((N, BH), jnp.float32)

    o = _sparse_matmul(xstack, rows2, cols_r, w_r, zeros)
    return jnp.concatenate([o[0], o[1]], axis=1).T
